# baseline (speedup 1.0000x reference)
import functools

import jax
import jax.numpy as jnp
from jax import lax
from jax.experimental import pallas as pl
from jax.experimental.pallas import tpu as pltpu
from jax.experimental.pallas import tpu_sc as plsc

_NV = 2562
_NF = 5120
_H, _W = 720, 1280
_HW = _H * _W
_C = 256
_CELLS = 90 * 160
_TBL = 15360

_NT = 32
_L = 16
_CHUNK = _HW // _NT
_FC = _NF // _NT
_NVP = 3072
_VC = _NVP // _NT

_mesh = plsc.VectorSubcoreMesh(core_axis_name="c", subcore_axis_name="s")
_sc_params = pltpu.CompilerParams(needs_layout_passes=False)


def _wid():
    return lax.axis_index("s") * 2 + lax.axis_index("c")


def _facemax_body(tri_hbm, out_hbm, tri_v, acc_v, red_v, sem):
    wid = _wid()
    base = wid * _CHUNK
    cp = pltpu.async_copy(tri_hbm.at[pl.ds(base, _CHUNK)], tri_v, sem)
    neg1 = jnp.full((_L,), -1, jnp.int32)

    def initb(i, carry):
        for u in range(8):
            acc_v[pl.ds((i * 8 + u) * _L, _L)] = neg1
        return carry

    lax.fori_loop(0, _L * _NF // (8 * _L), initb, 0)
    cp.wait()

    lanes = lax.iota(jnp.int32, _L)
    lane_off = lanes * _NF
    pix0 = base + lanes

    def body(i, carry):
        for u in range(8):
            j = i * 8 + u
            t = tri_v[pl.ds(j * _L, _L)]
            plsc.store_scatter(acc_v, [lane_off + t], pix0 + j * _L)
        return carry

    lax.fori_loop(0, _CHUNK // (8 * _L), body, 0)

    def redb(i, carry):
        m = acc_v[pl.ds(i * _L, _L)]
        for l in range(1, _L):
            m = jnp.maximum(m, acc_v[pl.ds(l * _NF + i * _L, _L)])
        red_v[pl.ds(i * _L, _L)] = m
        return carry

    lax.fori_loop(0, _NF // _L, redb, 0)
    pltpu.sync_copy(red_v, out_hbm.at[pl.ds(wid * _NF, _NF)])


_facemax_k = pl.kernel(
    _facemax_body,
    out_type=jax.ShapeDtypeStruct((_NT * _NF,), jnp.int32),
    mesh=_mesh,
    compiler_params=_sc_params,
    scratch_types=[
        pltpu.VMEM((_CHUNK,), jnp.int32),
        pltpu.VMEM((_L * _NF,), jnp.int32),
        pltpu.VMEM((_NF,), jnp.int32),
        pltpu.SemaphoreType.DMA,
    ],
)


def _vertkey_body(part_hbm, facesT_hbm, out_hbm, mbuf, fbuf, acc_v, red_v, sem):
    wid = _wid()
    fb = wid * _FC
    cps = []
    for r in range(_NT):
        cps.append(pltpu.async_copy(
            part_hbm.at[pl.ds(r * _NF + fb, _FC)],
            mbuf.at[pl.ds(r * _FC, _FC)], sem))
    for k in range(3):
        cps.append(pltpu.async_copy(
            facesT_hbm.at[pl.ds(k * _NF + fb, _FC)],
            fbuf.at[pl.ds(k * _FC, _FC)], sem))

    neg1 = jnp.full((_L,), -1, jnp.int32)

    def initb(i, carry):
        for u in range(8):
            acc_v[pl.ds((i * 8 + u) * _L, _L)] = neg1
        return carry

    lax.fori_loop(0, _L * _NVP // (8 * _L), initb, 0)
    for cp in cps:
        cp.wait()

    lanes = lax.iota(jnp.int32, _L)
    lane_off = lanes * _NVP

    def mainb(i, carry):
        m = mbuf[pl.ds(i * _L, _L)]
        for r in range(1, _NT):
            m = jnp.maximum(m, mbuf[pl.ds(r * _FC + i * _L, _L)])
        valid = m >= 0
        for k in range(3):
            vid = fbuf[pl.ds(k * _FC + i * _L, _L)]
            key = jnp.where(valid, k * _HW + m, -1)
            idx = lane_off + vid
            old = plsc.load_gather(acc_v, [idx])
            plsc.store_scatter(acc_v, [idx], jnp.maximum(old, key))
        return carry

    lax.fori_loop(0, _FC // _L, mainb, 0)

    def redb(i, carry):
        m = acc_v[pl.ds(i * _L, _L)]
        for l in range(1, _L):
            m = jnp.maximum(m, acc_v[pl.ds(l * _NVP + i * _L, _L)])
        red_v[pl.ds(i * _L, _L)] = m
        return carry

    lax.fori_loop(0, _NVP // _L, redb, 0)
    pltpu.sync_copy(red_v, out_hbm.at[pl.ds(wid * _NVP, _NVP)])


_vertkey_k = pl.kernel(
    _vertkey_body,
    out_type=jax.ShapeDtypeStruct((_NT * _NVP,), jnp.int32),
    mesh=_mesh,
    compiler_params=_sc_params,
    scratch_types=[
        pltpu.VMEM((_NT * _FC,), jnp.int32),
        pltpu.VMEM((3 * _FC,), jnp.int32),
        pltpu.VMEM((_L * _NVP,), jnp.int32),
        pltpu.VMEM((_NVP,), jnp.int32),
        pltpu.SemaphoreType.DMA,
    ],
)


def _gather_body(vpart_hbm, feat_hbm, vfeat_hbm, zvec_hbm,
                 kbuf, cells_v, zv, rows_v, sem):
    wid = _wid()
    vb = wid * _VC
    cps = []
    for r in range(_NT):
        cps.append(pltpu.async_copy(
            vpart_hbm.at[pl.ds(r * _NVP + vb, _VC)],
            kbuf.at[pl.ds(r * _VC, _VC)], sem))
    for cp in cps:
        cp.wait()

    def cb(i, carry):
        m = kbuf[pl.ds(i * _L, _L)]
        for r in range(1, _NT):
            m = jnp.maximum(m, kbuf[pl.ds(r * _VC + i * _L, _L)])
        valid = m >= 0
        pix = lax.rem(m, _HW)
        ii = lax.div(pix, _W)
        jj = lax.rem(pix, _W)
        cell = lax.div(ii, 8) * 160 + lax.div(jj, 8)
        cells_v[pl.ds(i * _L, _L)] = jnp.where(valid, cell, _CELLS)
        zv[pl.ds(i * _L, _L)] = jnp.where(valid, 0.0, 1.0)
        return carry

    lax.fori_loop(0, _VC // _L, cb, 0)

    cvecs = [cells_v[pl.ds(b * _L, _L)] for b in range(_VC // _L)]
    gs = []
    for i in range(_VC):
        cell = cvecs[i // _L][i % _L]
        gs.append(pltpu.async_copy(
            feat_hbm.at[cell], rows_v.at[i], sem))
    for g in gs:
        g.wait()
    pltpu.sync_copy(rows_v, vfeat_hbm.at[pl.ds(vb, _VC)])
    pltpu.sync_copy(zv, zvec_hbm.at[pl.ds(vb, _VC)])


_gather_k = pl.kernel(
    _gather_body,
    out_type=(
        jax.ShapeDtypeStruct((_NVP, _C), jnp.float32),
        jax.ShapeDtypeStruct((_NVP,), jnp.float32),
    ),
    mesh=_mesh,
    compiler_params=_sc_params,
    scratch_types=[
        pltpu.VMEM((_NT * _VC,), jnp.int32),
        pltpu.VMEM((_VC,), jnp.int32),
        pltpu.VMEM((_VC,), jnp.float32),
        pltpu.VMEM((_VC, _C), jnp.float32),
        pltpu.SemaphoreType.DMA,
    ],
)


def _transpose_body(x_ref, o_ref):
    i = pl.program_id(0)
    x = x_ref[...]
    xt = jnp.transpose(x)
    rows = lax.broadcasted_iota(jnp.int32, (1024, 1), 0) + i * 1024
    o_ref[...] = jnp.where(rows < _CELLS, xt, 0.0)


def _transpose_call(img2d):
    return pl.pallas_call(
        _transpose_body,
        grid=(_TBL // 1024,),
        in_specs=[pl.BlockSpec((_C, 1024), lambda i: (0, i))],
        out_specs=pl.BlockSpec((1024, _C), lambda i: (i, 0)),
        out_shape=jax.ShapeDtypeStruct((_TBL, _C), jnp.float32),
    )(img2d)


def _mask_body(z_ref, o_ref):
    o_ref[...] = jnp.broadcast_to(z_ref[...], o_ref.shape)


def _mask_call(zcol):
    return pl.pallas_call(
        _mask_body,
        grid=(pl.cdiv(_NV, 128),),
        in_specs=[pl.BlockSpec((128, 1), lambda i: (i, 0))],
        out_specs=pl.BlockSpec((128, _NV), lambda i: (i, 0)),
        out_shape=jax.ShapeDtypeStruct((_NV, _NV), jnp.float32),
    )(zcol)


def kernel(rgb_filename, vertices_mesh, faces_mesh, cam_extrinsics,
           intrinsics_mat, image_features, tri_ids):
    tri_flat = tri_ids.reshape(-1).astype(jnp.int32)
    facesT = jnp.transpose(faces_mesh[0]).reshape(-1).astype(jnp.int32)
    img2d = image_features.reshape(_C, _CELLS)

    partials = _facemax_k(tri_flat)
    feat_pad = _transpose_call(img2d)
    vpart = _vertkey_k(partials, facesT)
    vfeat, zvec = _gather_k(vpart, feat_pad)

    zcol = zvec[:_NV].reshape(_NV, 1)
    attn_mask = _mask_call(zcol)
    return attn_mask, vfeat[:_NV][None]


# speedup vs baseline: 1.0709x; 1.0709x over previous
import functools

import jax
import jax.numpy as jnp
from jax import lax
from jax.experimental import pallas as pl
from jax.experimental.pallas import tpu as pltpu
from jax.experimental.pallas import tpu_sc as plsc

_NV = 2562
_NF = 5120
_H, _W = 720, 1280
_HW = _H * _W
_C = 256
_CELLS = 90 * 160
_TBL = 15360

_NT = 32
_L = 16
_CHUNK = _HW // _NT
_FC = _NF // _NT
_NVP = 3072
_VC = _NVP // _NT

_mesh = plsc.VectorSubcoreMesh(core_axis_name="c", subcore_axis_name="s")
_sc_params = pltpu.CompilerParams(needs_layout_passes=False)


def _wid():
    return lax.axis_index("s") * 2 + lax.axis_index("c")


def _facemax_body(tri_hbm, out_hbm, tri_v, acc_v, red_v, sem):
    wid = _wid()
    base = wid * _CHUNK
    cp = pltpu.async_copy(tri_hbm.at[pl.ds(base, _CHUNK)], tri_v, sem)
    neg1 = jnp.full((_L,), -1, jnp.int32)

    def initb(i, carry):
        for u in range(8):
            acc_v[pl.ds((i * 8 + u) * _L, _L)] = neg1
        return carry

    lax.fori_loop(0, _L * _NF // (8 * _L), initb, 0)
    cp.wait()

    lanes = lax.iota(jnp.int32, _L)
    lane_off = lanes * _NF
    pix0 = base + lanes

    def body(i, carry):
        for u in range(8):
            j = i * 8 + u
            t = tri_v[pl.ds(j * _L, _L)]
            plsc.store_scatter(acc_v, [lane_off + t], pix0 + j * _L)
        return carry

    lax.fori_loop(0, _CHUNK // (8 * _L), body, 0)

    def redb(i, carry):
        m = acc_v[pl.ds(i * _L, _L)]
        for l in range(1, _L):
            m = jnp.maximum(m, acc_v[pl.ds(l * _NF + i * _L, _L)])
        red_v[pl.ds(i * _L, _L)] = m
        return carry

    lax.fori_loop(0, _NF // _L, redb, 0)
    pltpu.sync_copy(red_v, out_hbm.at[pl.ds(wid * _NF, _NF)])


_facemax_k = pl.kernel(
    _facemax_body,
    out_type=jax.ShapeDtypeStruct((_NT * _NF,), jnp.int32),
    mesh=_mesh,
    compiler_params=_sc_params,
    scratch_types=[
        pltpu.VMEM((_CHUNK,), jnp.int32),
        pltpu.VMEM((_L * _NF,), jnp.int32),
        pltpu.VMEM((_NF,), jnp.int32),
        pltpu.SemaphoreType.DMA,
    ],
)


def _vertkey_body(part_hbm, facesT_hbm, out_hbm, mbuf, fbuf, acc_v, red_v, sem):
    wid = _wid()
    fb = wid * _FC
    cps = []
    for r in range(_NT):
        cps.append(pltpu.async_copy(
            part_hbm.at[pl.ds(r * _NF + fb, _FC)],
            mbuf.at[pl.ds(r * _FC, _FC)], sem))
    for k in range(3):
        cps.append(pltpu.async_copy(
            facesT_hbm.at[pl.ds(k * _NF + fb, _FC)],
            fbuf.at[pl.ds(k * _FC, _FC)], sem))

    neg1 = jnp.full((_L,), -1, jnp.int32)

    def initb(i, carry):
        for u in range(8):
            acc_v[pl.ds((i * 8 + u) * _L, _L)] = neg1
        return carry

    lax.fori_loop(0, _L * _NVP // (8 * _L), initb, 0)
    for cp in cps:
        cp.wait()

    lanes = lax.iota(jnp.int32, _L)
    lane_off = lanes * _NVP

    def mainb(i, carry):
        m = mbuf[pl.ds(i * _L, _L)]
        for r in range(1, _NT):
            m = jnp.maximum(m, mbuf[pl.ds(r * _FC + i * _L, _L)])
        valid = m >= 0
        for k in range(3):
            vid = fbuf[pl.ds(k * _FC + i * _L, _L)]
            key = jnp.where(valid, k * _HW + m, -1)
            idx = lane_off + vid
            old = plsc.load_gather(acc_v, [idx])
            plsc.store_scatter(acc_v, [idx], jnp.maximum(old, key))
        return carry

    lax.fori_loop(0, _FC // _L, mainb, 0)

    def redb(i, carry):
        m = acc_v[pl.ds(i * _L, _L)]
        for l in range(1, _L):
            m = jnp.maximum(m, acc_v[pl.ds(l * _NVP + i * _L, _L)])
        red_v[pl.ds(i * _L, _L)] = m
        return carry

    lax.fori_loop(0, _NVP // _L, redb, 0)
    pltpu.sync_copy(red_v, out_hbm.at[pl.ds(wid * _NVP, _NVP)])


_vertkey_k = pl.kernel(
    _vertkey_body,
    out_type=jax.ShapeDtypeStruct((_NT * _NVP,), jnp.int32),
    mesh=_mesh,
    compiler_params=_sc_params,
    scratch_types=[
        pltpu.VMEM((_NT * _FC,), jnp.int32),
        pltpu.VMEM((3 * _FC,), jnp.int32),
        pltpu.VMEM((_L * _NVP,), jnp.int32),
        pltpu.VMEM((_NVP,), jnp.int32),
        pltpu.SemaphoreType.DMA,
    ],
)


def _attn_group_rows(stage, zvecs, i0, n_rows):
    fvecs = []
    for r in range(n_rows):
        i = i0 + r
        s = zvecs[i // _L][i % _L]
        fvecs.append(jnp.broadcast_to(s.astype(jnp.float32), (_L,)))

    def fill(c, carry):
        for r in range(n_rows):
            stage[r, pl.ds(c * _L, _L)] = fvecs[r]
        return carry

    lax.fori_loop(0, _NV // _L, fill, 0)
    tail = 2546 + lax.iota(jnp.int32, _L)
    for r in range(n_rows):
        plsc.store_scatter(stage, [jnp.full((_L,), r, jnp.int32), tail],
                           fvecs[r])


def _gather_body(vpart_hbm, feat_hbm, vfeat_hbm, attn_hbm,
                 kbuf, cells_v, zsel_v, stage, stage2, rows_v, sem):
    wid = _wid()
    vb = wid * _VC
    cps = []
    for r in range(_NT):
        cps.append(pltpu.async_copy(
            vpart_hbm.at[pl.ds(r * _NVP + vb, _VC)],
            kbuf.at[pl.ds(r * _VC, _VC)], sem))
    for cp in cps:
        cp.wait()

    def cb(i, carry):
        m = kbuf[pl.ds(i * _L, _L)]
        for r in range(1, _NT):
            m = jnp.maximum(m, kbuf[pl.ds(r * _VC + i * _L, _L)])
        valid = m >= 0
        pix = lax.rem(m, _HW)
        ii = lax.div(pix, _W)
        jj = lax.rem(pix, _W)
        cell = lax.div(ii, 8) * 160 + lax.div(jj, 8)
        cells_v[pl.ds(i * _L, _L)] = jnp.where(valid, cell, _CELLS)
        zsel_v[pl.ds(i * _L, _L)] = jnp.where(valid, 0, 1)
        return carry

    lax.fori_loop(0, _VC // _L, cb, 0)

    g = pltpu.async_copy(feat_hbm.at[cells_v], rows_v, sem)

    zvecs = [zsel_v[pl.ds(b * _L, _L)] for b in range(_VC // _L)]

    @pl.when(vb + _VC <= _NV)
    def _attn_full():
        for grp in range(_VC // 8):
            _attn_group_rows(stage, zvecs, 8 * grp, 8)
            pltpu.sync_copy(stage, attn_hbm.at[pl.ds(vb + 8 * grp, 8)])

    @pl.when(jnp.logical_and(vb < _NV, vb + _VC > _NV))
    def _attn_tail():
        for grp in range((_NV % _VC) // 8):
            _attn_group_rows(stage, zvecs, 8 * grp, 8)
            pltpu.sync_copy(stage, attn_hbm.at[pl.ds(vb + 8 * grp, 8)])
        _attn_group_rows(stage2, zvecs, (_NV % _VC) // 8 * 8, 2)
        pltpu.sync_copy(stage2, attn_hbm.at[pl.ds(_NV - 2, 2)])

    g.wait()
    pltpu.sync_copy(rows_v, vfeat_hbm.at[pl.ds(vb, _VC)])


_gather_k = pl.kernel(
    _gather_body,
    out_type=(
        jax.ShapeDtypeStruct((_NVP, _C), jnp.float32),
        jax.ShapeDtypeStruct((_NV, _NV), jnp.float32),
    ),
    mesh=_mesh,
    compiler_params=_sc_params,
    scratch_types=[
        pltpu.VMEM((_NT * _VC,), jnp.int32),
        pltpu.VMEM((_VC,), jnp.int32),
        pltpu.VMEM((_VC,), jnp.int32),
        pltpu.VMEM((8, _NV), jnp.float32),
        pltpu.VMEM((2, _NV), jnp.float32),
        pltpu.VMEM((_VC, _C), jnp.float32),
        pltpu.SemaphoreType.DMA,
    ],
)


def _transpose_body(x_ref, o_ref):
    i = pl.program_id(0)
    x = x_ref[...]
    xt = jnp.transpose(x)
    rows = lax.broadcasted_iota(jnp.int32, (1024, 1), 0) + i * 1024
    o_ref[...] = jnp.where(rows < _CELLS, xt, 0.0)


def _transpose_call(img2d):
    return pl.pallas_call(
        _transpose_body,
        grid=(_TBL // 1024,),
        in_specs=[pl.BlockSpec((_C, 1024), lambda i: (0, i))],
        out_specs=pl.BlockSpec((1024, _C), lambda i: (i, 0)),
        out_shape=jax.ShapeDtypeStruct((_TBL, _C), jnp.float32),
    )(img2d)


def kernel(rgb_filename, vertices_mesh, faces_mesh, cam_extrinsics,
           intrinsics_mat, image_features, tri_ids):
    tri_flat = tri_ids.reshape(-1).astype(jnp.int32)
    facesT = jnp.transpose(faces_mesh[0]).reshape(-1).astype(jnp.int32)
    img2d = image_features.reshape(_C, _CELLS)

    partials = _facemax_k(tri_flat)
    feat_pad = _transpose_call(img2d)
    vpart = _vertkey_k(partials, facesT)
    vfeat, attn_mask = _gather_k(vpart, feat_pad)

    return attn_mask, vfeat[:_NV][None]


# speedup vs baseline: 1.0744x; 1.0033x over previous
import functools

import jax
import jax.numpy as jnp
from jax import lax
from jax.experimental import pallas as pl
from jax.experimental.pallas import tpu as pltpu
from jax.experimental.pallas import tpu_sc as plsc

_NV = 2562
_NF = 5120
_H, _W = 720, 1280
_HW = _H * _W
_C = 256
_CELLS = 90 * 160
_TBL = 15360

_NT = 32
_L = 16
_CHUNK = _HW // _NT
_FC = _NF // _NT
_NVP = 3072
_VC = _NVP // _NT

_mesh = plsc.VectorSubcoreMesh(core_axis_name="c", subcore_axis_name="s")
_sc_params = pltpu.CompilerParams(needs_layout_passes=False)


def _wid():
    return lax.axis_index("s") * 2 + lax.axis_index("c")


def _facemax_body(tri_hbm, out_hbm, tri_v, acc_v, red_v, sem):
    wid = _wid()
    base = wid * _CHUNK
    cp = pltpu.async_copy(tri_hbm.at[pl.ds(base, _CHUNK)], tri_v, sem)
    neg1 = jnp.full((_L,), -1, jnp.int32)

    def initb(i, carry):
        for u in range(8):
            acc_v[pl.ds((i * 8 + u) * _L, _L)] = neg1
        return carry

    lax.fori_loop(0, _L * _NF // (8 * _L), initb, 0)
    cp.wait()

    lanes = lax.iota(jnp.int32, _L)
    lane_off = lanes * _NF
    pix0 = base + lanes

    def body(i, carry):
        for u in range(8):
            j = i * 8 + u
            t = tri_v[pl.ds(j * _L, _L)]
            plsc.store_scatter(acc_v, [lane_off + t], pix0 + j * _L)
        return carry

    lax.fori_loop(0, _CHUNK // (8 * _L), body, 0)

    def redb(i, carry):
        m = acc_v[pl.ds(i * _L, _L)]
        for l in range(1, _L):
            m = jnp.maximum(m, acc_v[pl.ds(l * _NF + i * _L, _L)])
        red_v[pl.ds(i * _L, _L)] = m
        return carry

    lax.fori_loop(0, _NF // _L, redb, 0)
    pltpu.sync_copy(red_v, out_hbm.at[pl.ds(wid * _NF, _NF)])


_facemax_k = pl.kernel(
    _facemax_body,
    out_type=jax.ShapeDtypeStruct((_NT * _NF,), jnp.int32),
    mesh=_mesh,
    compiler_params=_sc_params,
    scratch_types=[
        pltpu.VMEM((_CHUNK,), jnp.int32),
        pltpu.VMEM((_L * _NF,), jnp.int32),
        pltpu.VMEM((_NF,), jnp.int32),
        pltpu.SemaphoreType.DMA,
    ],
)


def _vertkey_body(part_hbm, facesT_hbm, out_hbm, mbuf, fbuf, acc_v, red_v, sem):
    wid = _wid()
    fb = wid * _FC
    cps = []
    for r in range(_NT):
        cps.append(pltpu.async_copy(
            part_hbm.at[pl.ds(r * _NF + fb, _FC)],
            mbuf.at[pl.ds(r * _FC, _FC)], sem))
    for k in range(3):
        cps.append(pltpu.async_copy(
            facesT_hbm.at[pl.ds(k * _NF + fb, _FC)],
            fbuf.at[pl.ds(k * _FC, _FC)], sem))

    neg1 = jnp.full((_L,), -1, jnp.int32)

    def initb(i, carry):
        for u in range(8):
            acc_v[pl.ds((i * 8 + u) * _L, _L)] = neg1
        return carry

    lax.fori_loop(0, _L * _NVP // (8 * _L), initb, 0)
    for cp in cps:
        cp.wait()

    lanes = lax.iota(jnp.int32, _L)
    lane_off = lanes * _NVP

    def mainb(i, carry):
        m = mbuf[pl.ds(i * _L, _L)]
        for r in range(1, _NT):
            m = jnp.maximum(m, mbuf[pl.ds(r * _FC + i * _L, _L)])
        valid = m >= 0
        for k in range(3):
            vid = fbuf[pl.ds(k * _FC + i * _L, _L)]
            key = jnp.where(valid, k * _HW + m, -1)
            idx = lane_off + vid
            old = plsc.load_gather(acc_v, [idx])
            plsc.store_scatter(acc_v, [idx], jnp.maximum(old, key))
        return carry

    lax.fori_loop(0, _FC // _L, mainb, 0)

    def redb(i, carry):
        m = acc_v[pl.ds(i * _L, _L)]
        for l in range(1, _L):
            m = jnp.maximum(m, acc_v[pl.ds(l * _NVP + i * _L, _L)])
        red_v[pl.ds(i * _L, _L)] = m
        return carry

    lax.fori_loop(0, _NVP // _L, redb, 0)
    pltpu.sync_copy(red_v, out_hbm.at[pl.ds(wid * _NVP, _NVP)])


_vertkey_k = pl.kernel(
    _vertkey_body,
    out_type=jax.ShapeDtypeStruct((_NT * _NVP,), jnp.int32),
    mesh=_mesh,
    compiler_params=_sc_params,
    scratch_types=[
        pltpu.VMEM((_NT * _FC,), jnp.int32),
        pltpu.VMEM((3 * _FC,), jnp.int32),
        pltpu.VMEM((_L * _NVP,), jnp.int32),
        pltpu.VMEM((_NVP,), jnp.int32),
        pltpu.SemaphoreType.DMA,
    ],
)


def _attn_group_rows(stage, zvecs, i0, n_rows):
    fvecs = []
    for r in range(n_rows):
        i = i0 + r
        s = zvecs[i // _L][i % _L]
        fvecs.append(jnp.broadcast_to(s.astype(jnp.float32), (_L,)))

    def fill(c, carry):
        for r in range(n_rows):
            stage[r, pl.ds(c * _L, _L)] = fvecs[r]
        return carry

    lax.fori_loop(0, _NV // _L, fill, 0)
    tail = 2546 + lax.iota(jnp.int32, _L)
    for r in range(n_rows):
        plsc.store_scatter(stage, [jnp.full((_L,), r, jnp.int32), tail],
                           fvecs[r])


def _gather_body(vpart_hbm, feat_hbm, vfeat_hbm, attn_hbm,
                 kbuf, cells_v, zsel_v, stage, stage_b, stage2, rows_v,
                 sem, asem_a, asem_b):
    wid = _wid()
    vb = wid * _VC
    cps = []
    for r in range(_NT):
        cps.append(pltpu.async_copy(
            vpart_hbm.at[pl.ds(r * _NVP + vb, _VC)],
            kbuf.at[pl.ds(r * _VC, _VC)], sem))
    for cp in cps:
        cp.wait()

    def cb(i, carry):
        m = kbuf[pl.ds(i * _L, _L)]
        for r in range(1, _NT):
            m = jnp.maximum(m, kbuf[pl.ds(r * _VC + i * _L, _L)])
        valid = m >= 0
        pix = lax.rem(m, _HW)
        ii = lax.div(pix, _W)
        jj = lax.rem(pix, _W)
        cell = lax.div(ii, 8) * 160 + lax.div(jj, 8)
        cells_v[pl.ds(i * _L, _L)] = jnp.where(valid, cell, _CELLS)
        zsel_v[pl.ds(i * _L, _L)] = jnp.where(valid, 0, 1)
        return carry

    lax.fori_loop(0, _VC // _L, cb, 0)

    g = pltpu.async_copy(feat_hbm.at[cells_v], rows_v, sem)

    zvecs = [zsel_v[pl.ds(b * _L, _L)] for b in range(_VC // _L)]

    @pl.when(vb + _VC <= _NV)
    def _attn_full():
        stages = (stage, stage_b)
        sems = (asem_a, asem_b)
        pend = [None, None]
        for grp in range(_VC // 8):
            k = grp % 2
            if pend[k] is not None:
                pend[k].wait()
            _attn_group_rows(stages[k], zvecs, 8 * grp, 8)
            pend[k] = pltpu.async_copy(
                stages[k], attn_hbm.at[pl.ds(vb + 8 * grp, 8)], sems[k])
        pend[0].wait()
        pend[1].wait()

    @pl.when(jnp.logical_and(vb < _NV, vb + _VC > _NV))
    def _attn_tail():
        for grp in range((_NV % _VC) // 8):
            _attn_group_rows(stage, zvecs, 8 * grp, 8)
            pltpu.sync_copy(stage, attn_hbm.at[pl.ds(vb + 8 * grp, 8)])
        _attn_group_rows(stage2, zvecs, (_NV % _VC) // 8 * 8, 2)
        pltpu.sync_copy(stage2, attn_hbm.at[pl.ds(_NV - 2, 2)])

    g.wait()
    pltpu.sync_copy(rows_v, vfeat_hbm.at[pl.ds(vb, _VC)])


_gather_k = pl.kernel(
    _gather_body,
    out_type=(
        jax.ShapeDtypeStruct((_NVP, _C), jnp.float32),
        jax.ShapeDtypeStruct((_NV, _NV), jnp.float32),
    ),
    mesh=_mesh,
    compiler_params=_sc_params,
    scratch_types=[
        pltpu.VMEM((_NT * _VC,), jnp.int32),
        pltpu.VMEM((_VC,), jnp.int32),
        pltpu.VMEM((_VC,), jnp.int32),
        pltpu.VMEM((8, _NV), jnp.float32),
        pltpu.VMEM((8, _NV), jnp.float32),
        pltpu.VMEM((2, _NV), jnp.float32),
        pltpu.VMEM((_VC, _C), jnp.float32),
        pltpu.SemaphoreType.DMA,
        pltpu.SemaphoreType.DMA,
        pltpu.SemaphoreType.DMA,
    ],
)


def _transpose_body(x_ref, o_ref):
    i = pl.program_id(0)
    x = x_ref[...]
    xt = jnp.transpose(x)
    rows = lax.broadcasted_iota(jnp.int32, (1024, 1), 0) + i * 1024
    o_ref[...] = jnp.where(rows < _CELLS, xt, 0.0)


def _transpose_call(img2d):
    return pl.pallas_call(
        _transpose_body,
        grid=(_TBL // 1024,),
        in_specs=[pl.BlockSpec((_C, 1024), lambda i: (0, i))],
        out_specs=pl.BlockSpec((1024, _C), lambda i: (i, 0)),
        out_shape=jax.ShapeDtypeStruct((_TBL, _C), jnp.float32),
    )(img2d)


def kernel(rgb_filename, vertices_mesh, faces_mesh, cam_extrinsics,
           intrinsics_mat, image_features, tri_ids):
    tri_flat = tri_ids.reshape(-1).astype(jnp.int32)
    facesT = jnp.transpose(faces_mesh[0]).reshape(-1).astype(jnp.int32)
    img2d = image_features.reshape(_C, _CELLS)

    partials = _facemax_k(tri_flat)
    feat_pad = _transpose_call(img2d)
    vpart = _vertkey_k(partials, facesT)
    vfeat, attn_mask = _gather_k(vpart, feat_pad)

    return attn_mask, vfeat[:_NV][None]


# speedup vs baseline: 1.0756x; 1.0010x over previous
import functools

import jax
import jax.numpy as jnp
from jax import lax
from jax.experimental import pallas as pl
from jax.experimental.pallas import tpu as pltpu
from jax.experimental.pallas import tpu_sc as plsc

_NV = 2562
_NF = 5120
_H, _W = 720, 1280
_HW = _H * _W
_C = 256
_CELLS = 90 * 160
_TBL = 15360

_NT = 32
_L = 16
_CHUNK = _HW // _NT
_FC = _NF // _NT
_NVP = 3072
_VC = _NVP // _NT

_mesh = plsc.VectorSubcoreMesh(core_axis_name="c", subcore_axis_name="s")
_sc_params = pltpu.CompilerParams(needs_layout_passes=False)


def _wid():
    return lax.axis_index("s") * 2 + lax.axis_index("c")


def _facemax_body(tri_hbm, out_hbm, tri_v, acc_v, red_v, sem):
    wid = _wid()
    base = wid * _CHUNK
    cp = pltpu.async_copy(tri_hbm.at[pl.ds(base, _CHUNK)], tri_v, sem)
    neg1 = jnp.full((_L,), -1, jnp.int32)

    def initb(i, carry):
        for u in range(8):
            acc_v[pl.ds((i * 8 + u) * _L, _L)] = neg1
        return carry

    lax.fori_loop(0, _L * _NF // (8 * _L), initb, 0)
    cp.wait()

    lanes = lax.iota(jnp.int32, _L)
    lane_off = lanes * _NF
    pix0 = base + lanes

    def body(i, carry):
        for u in range(8):
            j = i * 8 + u
            t = tri_v[pl.ds(j * _L, _L)]
            plsc.store_scatter(acc_v, [lane_off + t], pix0 + j * _L)
        return carry

    lax.fori_loop(0, _CHUNK // (8 * _L), body, 0)

    def redb(i, carry):
        m = acc_v[pl.ds(i * _L, _L)]
        for l in range(1, _L):
            m = jnp.maximum(m, acc_v[pl.ds(l * _NF + i * _L, _L)])
        red_v[pl.ds(i * _L, _L)] = m
        return carry

    lax.fori_loop(0, _NF // _L, redb, 0)
    pltpu.sync_copy(red_v, out_hbm.at[pl.ds(wid * _NF, _NF)])


_facemax_k = pl.kernel(
    _facemax_body,
    out_type=jax.ShapeDtypeStruct((_NT * _NF,), jnp.int32),
    mesh=_mesh,
    compiler_params=_sc_params,
    scratch_types=[
        pltpu.VMEM((_CHUNK,), jnp.int32),
        pltpu.VMEM((_L * _NF,), jnp.int32),
        pltpu.VMEM((_NF,), jnp.int32),
        pltpu.SemaphoreType.DMA,
    ],
)


def _vertkey_body(part_hbm, facesT_hbm, out_hbm, mbuf, fbuf, acc_v, red_v, sem):
    wid = _wid()
    fb = wid * _FC
    cps = []
    for r in range(_NT):
        cps.append(pltpu.async_copy(
            part_hbm.at[pl.ds(r * _NF + fb, _FC)],
            mbuf.at[pl.ds(r * _FC, _FC)], sem))
    for k in range(3):
        cps.append(pltpu.async_copy(
            facesT_hbm.at[pl.ds(k * _NF + fb, _FC)],
            fbuf.at[pl.ds(k * _FC, _FC)], sem))

    neg1 = jnp.full((_L,), -1, jnp.int32)

    def initb(i, carry):
        for u in range(8):
            acc_v[pl.ds((i * 8 + u) * _L, _L)] = neg1
        return carry

    lax.fori_loop(0, _L * _NVP // (8 * _L), initb, 0)
    for cp in cps:
        cp.wait()

    lanes = lax.iota(jnp.int32, _L)
    lane_off = lanes * _NVP

    def mainb(i, carry):
        m = mbuf[pl.ds(i * _L, _L)]
        for r in range(1, _NT):
            m = jnp.maximum(m, mbuf[pl.ds(r * _FC + i * _L, _L)])
        valid = m >= 0
        for k in range(3):
            vid = fbuf[pl.ds(k * _FC + i * _L, _L)]
            key = jnp.where(valid, k * _HW + m, -1)
            idx = lane_off + vid
            old = plsc.load_gather(acc_v, [idx])
            plsc.store_scatter(acc_v, [idx], jnp.maximum(old, key))
        return carry

    lax.fori_loop(0, _FC // _L, mainb, 0)

    def redb(i, carry):
        m = acc_v[pl.ds(i * _L, _L)]
        for l in range(1, _L):
            m = jnp.maximum(m, acc_v[pl.ds(l * _NVP + i * _L, _L)])
        red_v[pl.ds(i * _L, _L)] = m
        return carry

    lax.fori_loop(0, _NVP // _L, redb, 0)
    pltpu.sync_copy(red_v, out_hbm.at[pl.ds(wid * _NVP, _NVP)])


_vertkey_k = pl.kernel(
    _vertkey_body,
    out_type=jax.ShapeDtypeStruct((_NT * _NVP,), jnp.int32),
    mesh=_mesh,
    compiler_params=_sc_params,
    scratch_types=[
        pltpu.VMEM((_NT * _FC,), jnp.int32),
        pltpu.VMEM((3 * _FC,), jnp.int32),
        pltpu.VMEM((_L * _NVP,), jnp.int32),
        pltpu.VMEM((_NVP,), jnp.int32),
        pltpu.SemaphoreType.DMA,
    ],
)


def _attn_group_rows(stage, zvecs, i0, n_rows):
    fvecs = []
    for r in range(n_rows):
        i = i0 + r
        s = zvecs[i // _L][i % _L]
        fvecs.append(jnp.broadcast_to(s.astype(jnp.float32), (_L,)))

    def fill(c, carry):
        for r in range(n_rows):
            stage[r, pl.ds(c * _L, _L)] = fvecs[r]
        return carry

    lax.fori_loop(0, _NV // _L, fill, 0)
    tail = 2546 + lax.iota(jnp.int32, _L)
    for r in range(n_rows):
        plsc.store_scatter(stage, [jnp.full((_L,), r, jnp.int32), tail],
                           fvecs[r])


def _gather_body(vpart_hbm, feat_hbm, vfeat_hbm, attn_hbm,
                 kbuf, cells_v, zsel_v, stage, stage_b, stage2, rows_v,
                 sem, asem_a, asem_b):
    wid = _wid()
    vb = wid * _VC
    cps = []
    for r in range(_NT):
        cps.append(pltpu.async_copy(
            vpart_hbm.at[pl.ds(r * _NVP + vb, _VC)],
            kbuf.at[pl.ds(r * _VC, _VC)], sem))
    for cp in cps:
        cp.wait()

    def cb(i, carry):
        m = kbuf[pl.ds(i * _L, _L)]
        for r in range(1, _NT):
            m = jnp.maximum(m, kbuf[pl.ds(r * _VC + i * _L, _L)])
        valid = m >= 0
        pix = lax.rem(m, _HW)
        ii = lax.div(pix, _W)
        jj = lax.rem(pix, _W)
        cell = lax.div(ii, 8) * 160 + lax.div(jj, 8)
        cells_v[pl.ds(i * _L, _L)] = jnp.where(valid, cell, _CELLS)
        zsel_v[pl.ds(i * _L, _L)] = jnp.where(valid, 0, 1)
        return carry

    lax.fori_loop(0, _VC // _L, cb, 0)

    g = pltpu.async_copy(feat_hbm.at[cells_v], rows_v, sem)

    zvecs = [zsel_v[pl.ds(b * _L, _L)] for b in range(_VC // _L)]

    @pl.when(vb + _VC <= _NV)
    def _attn_full():
        stages = (stage, stage_b)
        sems = (asem_a, asem_b)
        pend = [None, None]
        for grp in range(_VC // 8):
            k = grp % 2
            if pend[k] is not None:
                pend[k].wait()
            _attn_group_rows(stages[k], zvecs, 8 * grp, 8)
            pend[k] = pltpu.async_copy(
                stages[k], attn_hbm.at[pl.ds(vb + 8 * grp, 8)], sems[k])
        pend[0].wait()
        pend[1].wait()

    @pl.when(jnp.logical_and(vb < _NV, vb + _VC > _NV))
    def _attn_tail():
        for grp in range((_NV % _VC) // 8):
            _attn_group_rows(stage, zvecs, 8 * grp, 8)
            pltpu.sync_copy(stage, attn_hbm.at[pl.ds(vb + 8 * grp, 8)])
        _attn_group_rows(stage2, zvecs, (_NV % _VC) // 8 * 8, 2)
        pltpu.sync_copy(stage2, attn_hbm.at[pl.ds(_NV - 2, 2)])

    g.wait()
    pltpu.sync_copy(rows_v, vfeat_hbm.at[pl.ds(vb, _VC)])


_gather_k = pl.kernel(
    _gather_body,
    out_type=(
        jax.ShapeDtypeStruct((_NVP, _C), jnp.float32),
        jax.ShapeDtypeStruct((_NV, _NV), jnp.float32),
    ),
    mesh=_mesh,
    compiler_params=_sc_params,
    scratch_types=[
        pltpu.VMEM((_NT * _VC,), jnp.int32),
        pltpu.VMEM((_VC,), jnp.int32),
        pltpu.VMEM((_VC,), jnp.int32),
        pltpu.VMEM((8, _NV), jnp.float32),
        pltpu.VMEM((8, _NV), jnp.float32),
        pltpu.VMEM((2, _NV), jnp.float32),
        pltpu.VMEM((_VC, _C), jnp.float32),
        pltpu.SemaphoreType.DMA,
        pltpu.SemaphoreType.DMA,
        pltpu.SemaphoreType.DMA,
    ],
)


def _transpose_body(x_ref, o_ref):
    i = pl.program_id(0)
    x = x_ref[...]
    xt = jnp.transpose(x)
    rows = lax.broadcasted_iota(jnp.int32, (1024, 1), 0) + i * 1024
    o_ref[...] = jnp.where(rows < _CELLS, xt, 0.0)


def _transpose_call(img2d):
    return pl.pallas_call(
        _transpose_body,
        grid=(_TBL // 1024,),
        in_specs=[pl.BlockSpec((_C, 1024), lambda i: (0, i))],
        out_specs=pl.BlockSpec((1024, _C), lambda i: (i, 0)),
        out_shape=jax.ShapeDtypeStruct((_TBL, _C), jnp.float32),
    )(img2d)


def kernel(rgb_filename, vertices_mesh, faces_mesh, cam_extrinsics,
           intrinsics_mat, image_features, tri_ids):
    tri_flat = tri_ids.reshape(-1).astype(jnp.int32)
    facesT = jnp.transpose(faces_mesh[0]).reshape(-1).astype(jnp.int32)
    img2d = image_features.reshape(_C, _CELLS)

    partials = _facemax_k(tri_flat)
    vpart = _vertkey_k(partials, facesT)
    feat_pad = _transpose_call(img2d)
    vfeat, attn_mask = _gather_k(vpart, feat_pad)

    return attn_mask, vfeat[:_NV][None]


# speedup vs baseline: 1.2143x; 1.1290x over previous
import functools

import jax
import jax.numpy as jnp
from jax import lax
from jax.experimental import pallas as pl
from jax.experimental.pallas import tpu as pltpu
from jax.experimental.pallas import tpu_sc as plsc

_NV = 2562
_NF = 5120
_H, _W = 720, 1280
_HW = _H * _W
_C = 256
_CELLS = 90 * 160
_TBL = 15360

_NT = 32
_L = 16
_CHUNK = _HW // _NT
_FC = _NF // _NT
_NVP = 3072
_VC = _NVP // _NT

_mesh = plsc.VectorSubcoreMesh(core_axis_name="c", subcore_axis_name="s")
_sc_params = pltpu.CompilerParams(needs_layout_passes=False)


def _wid():
    return lax.axis_index("s") * 2 + lax.axis_index("c")


def _facemax_body(tri_hbm, out_hbm, tri_v, acc_v, red_v, sem):
    wid = _wid()
    base = wid * _CHUNK
    cp = pltpu.async_copy(tri_hbm.at[pl.ds(base, _CHUNK)], tri_v, sem)
    neg1 = jnp.full((_L,), -1, jnp.int32)

    def initb(i, carry):
        for u in range(8):
            acc_v[pl.ds((i * 8 + u) * _L, _L)] = neg1
        return carry

    lax.fori_loop(0, _L * _NF // (8 * _L), initb, 0)
    cp.wait()

    lanes = lax.iota(jnp.int32, _L)
    lane_off = lanes * _NF
    pix0 = base + lanes

    def body(i, carry):
        for u in range(8):
            j = i * 8 + u
            t = tri_v[pl.ds(j * _L, _L)]
            plsc.store_scatter(acc_v, [lane_off + t], pix0 + j * _L)
        return carry

    lax.fori_loop(0, _CHUNK // (8 * _L), body, 0)

    def redb(i, carry):
        m = acc_v[pl.ds(i * _L, _L)]
        for l in range(1, _L):
            m = jnp.maximum(m, acc_v[pl.ds(l * _NF + i * _L, _L)])
        red_v[pl.ds(i * _L, _L)] = m
        return carry

    lax.fori_loop(0, _NF // _L, redb, 0)
    pltpu.sync_copy(red_v, out_hbm.at[pl.ds(wid * _NF, _NF)])


_facemax_k = pl.kernel(
    _facemax_body,
    out_type=jax.ShapeDtypeStruct((_NT * _NF,), jnp.int32),
    mesh=_mesh,
    compiler_params=_sc_params,
    scratch_types=[
        pltpu.VMEM((_CHUNK,), jnp.int32),
        pltpu.VMEM((_L * _NF,), jnp.int32),
        pltpu.VMEM((_NF,), jnp.int32),
        pltpu.SemaphoreType.DMA,
    ],
)


_FCS = _NF // 16
def _attn_group_rows(stage, zvecs, i0, n_rows):
    fvecs = []
    for r in range(n_rows):
        i = i0 + r
        s = zvecs[i // _L][i % _L]
        fvecs.append(jnp.broadcast_to(s.astype(jnp.float32), (_L,)))

    def fill(c, carry):
        for r in range(n_rows):
            stage[r, pl.ds(c * _L, _L)] = fvecs[r]
        return carry

    lax.fori_loop(0, _NV // _L, fill, 0)
    tail = 2546 + lax.iota(jnp.int32, _L)
    for r in range(n_rows):
        plsc.store_scatter(stage, [jnp.full((_L,), r, jnp.int32), tail],
                           fvecs[r])


def _gather_body(part_hbm, facesT_hbm, feat_hbm, vfeat_hbm, attn_hbm,
                 mbuf, fbuf, acc_v, red_v, kbuf, cells_v, zsel_v,
                 stage, stage2, rows_v, shared, sem):
    s_id = lax.axis_index("s")
    wid = _wid()
    vb = wid * _VC
    fb = s_id * _FCS
    cps = []
    for r in range(_NT):
        cps.append(pltpu.async_copy(
            part_hbm.at[pl.ds(r * _NF + fb, _FCS)],
            mbuf.at[pl.ds(r * _FCS, _FCS)], sem))
    for k in range(3):
        cps.append(pltpu.async_copy(
            facesT_hbm.at[pl.ds(k * _NF + fb, _FCS)],
            fbuf.at[pl.ds(k * _FCS, _FCS)], sem))

    neg1 = jnp.full((_L,), -1, jnp.int32)

    def initb(i, carry):
        for u in range(8):
            acc_v[pl.ds((i * 8 + u) * _L, _L)] = neg1
        return carry

    lax.fori_loop(0, _L * _NVP // (8 * _L), initb, 0)
    for cp in cps:
        cp.wait()

    lanes = lax.iota(jnp.int32, _L)
    lane_off = lanes * _NVP

    def mainb(i, carry):
        m = mbuf[pl.ds(i * _L, _L)]
        for r in range(1, _NT):
            m = jnp.maximum(m, mbuf[pl.ds(r * _FCS + i * _L, _L)])
        valid = m >= 0
        for k in range(3):
            vid = fbuf[pl.ds(k * _FCS + i * _L, _L)]
            key = jnp.where(valid, k * _HW + m, -1)
            idx = lane_off + vid
            old = plsc.load_gather(acc_v, [idx])
            plsc.store_scatter(acc_v, [idx], jnp.maximum(old, key))
        return carry

    lax.fori_loop(0, _FCS // _L, mainb, 0)

    def redb(i, carry):
        m = acc_v[pl.ds(i * _L, _L)]
        for l in range(1, _L):
            m = jnp.maximum(m, acc_v[pl.ds(l * _NVP + i * _L, _L)])
        red_v[pl.ds(i * _L, _L)] = m
        return carry

    lax.fori_loop(0, _NVP // _L, redb, 0)

    pltpu.sync_copy(red_v, shared.at[pl.ds(s_id * _NVP, _NVP)])
    plsc.subcore_barrier()

    cps2 = []
    for t in range(16):
        cps2.append(pltpu.async_copy(
            shared.at[pl.ds(t * _NVP + vb, _VC)],
            kbuf.at[pl.ds(t * _VC, _VC)], sem))
    for cp in cps2:
        cp.wait()

    def cb(i, carry):
        m = kbuf[pl.ds(i * _L, _L)]
        for r in range(1, 16):
            m = jnp.maximum(m, kbuf[pl.ds(r * _VC + i * _L, _L)])
        valid = m >= 0
        pix = lax.rem(m, _HW)
        ii = lax.div(pix, _W)
        jj = lax.rem(pix, _W)
        cell = lax.div(ii, 8) * 160 + lax.div(jj, 8)
        cells_v[pl.ds(i * _L, _L)] = jnp.where(valid, cell, _CELLS)
        zsel_v[pl.ds(i * _L, _L)] = jnp.where(valid, 0, 1)
        return carry

    lax.fori_loop(0, _VC // _L, cb, 0)

    g = pltpu.async_copy(feat_hbm.at[cells_v], rows_v, sem)

    zvecs = [zsel_v[pl.ds(b * _L, _L)] for b in range(_VC // _L)]

    @pl.when(vb + _VC <= _NV)
    def _attn_full():
        for grp in range(_VC // 8):
            _attn_group_rows(stage, zvecs, 8 * grp, 8)
            pltpu.sync_copy(stage, attn_hbm.at[pl.ds(vb + 8 * grp, 8)])

    @pl.when(jnp.logical_and(vb < _NV, vb + _VC > _NV))
    def _attn_tail():
        for grp in range((_NV % _VC) // 8):
            _attn_group_rows(stage, zvecs, 8 * grp, 8)
            pltpu.sync_copy(stage, attn_hbm.at[pl.ds(vb + 8 * grp, 8)])
        _attn_group_rows(stage2, zvecs, (_NV % _VC) // 8 * 8, 2)
        pltpu.sync_copy(stage2, attn_hbm.at[pl.ds(_NV - 2, 2)])

    g.wait()
    pltpu.sync_copy(rows_v, vfeat_hbm.at[pl.ds(vb, _VC)])


_gather_k = pl.kernel(
    _gather_body,
    out_type=(
        jax.ShapeDtypeStruct((_NVP, _C), jnp.float32),
        jax.ShapeDtypeStruct((_NV, _NV), jnp.float32),
    ),
    mesh=_mesh,
    compiler_params=_sc_params,
    scratch_types=[
        pltpu.VMEM((_NT * _FCS,), jnp.int32),
        pltpu.VMEM((3 * _FCS,), jnp.int32),
        pltpu.VMEM((_L * _NVP,), jnp.int32),
        pltpu.VMEM((_NVP,), jnp.int32),
        pltpu.VMEM((16 * _VC,), jnp.int32),
        pltpu.VMEM((_VC,), jnp.int32),
        pltpu.VMEM((_VC,), jnp.int32),
        pltpu.VMEM((8, _NV), jnp.float32),
        pltpu.VMEM((2, _NV), jnp.float32),
        pltpu.VMEM((_VC, _C), jnp.float32),
        pltpu.VMEM_SHARED((16 * _NVP,), jnp.int32),
        pltpu.SemaphoreType.DMA,
    ],
)


def _transpose_body(x_ref, o_ref):
    i = pl.program_id(0)
    x = x_ref[...]
    xt = jnp.transpose(x)
    rows = lax.broadcasted_iota(jnp.int32, (1024, 1), 0) + i * 1024
    o_ref[...] = jnp.where(rows < _CELLS, xt, 0.0)


def _transpose_call(img2d):
    return pl.pallas_call(
        _transpose_body,
        grid=(_TBL // 1024,),
        in_specs=[pl.BlockSpec((_C, 1024), lambda i: (0, i))],
        out_specs=pl.BlockSpec((1024, _C), lambda i: (i, 0)),
        out_shape=jax.ShapeDtypeStruct((_TBL, _C), jnp.float32),
    )(img2d)


def kernel(rgb_filename, vertices_mesh, faces_mesh, cam_extrinsics,
           intrinsics_mat, image_features, tri_ids):
    tri_flat = tri_ids.reshape(-1).astype(jnp.int32)
    facesT = jnp.transpose(faces_mesh[0]).reshape(-1).astype(jnp.int32)
    img2d = image_features.reshape(_C, _CELLS)

    partials = _facemax_k(tri_flat)
    feat_pad = _transpose_call(img2d)
    vfeat, attn_mask = _gather_k(partials, facesT, feat_pad)

    return attn_mask, vfeat[:_NV][None]


# speedup vs baseline: 1.2550x; 1.0335x over previous
import functools

import jax
import jax.numpy as jnp
from jax import lax
from jax.experimental import pallas as pl
from jax.experimental.pallas import tpu as pltpu
from jax.experimental.pallas import tpu_sc as plsc

_NV = 2562
_NF = 5120
_H, _W = 720, 1280
_HW = _H * _W
_C = 256
_CELLS = 90 * 160
_TBL = 15360

_NT = 32
_L = 16
_CHUNK = _HW // _NT
_FC = _NF // _NT
_NVP = 3072
_VC = _NVP // _NT

_mesh = plsc.VectorSubcoreMesh(core_axis_name="c", subcore_axis_name="s")
_sc_params = pltpu.CompilerParams(needs_layout_passes=False)


def _wid():
    return lax.axis_index("s") * 2 + lax.axis_index("c")


def _facemax_body(tri_hbm, out_hbm, tri_v, acc_v, red_v, sem):
    wid = _wid()
    base = wid * _CHUNK
    cp = pltpu.async_copy(tri_hbm.at[pl.ds(base, _CHUNK)], tri_v, sem)
    neg1 = jnp.full((_L,), -1, jnp.int32)

    def initb(i, carry):
        for u in range(8):
            acc_v[pl.ds((i * 8 + u) * _L, _L)] = neg1
        return carry

    lax.fori_loop(0, _L * _NF // (8 * _L), initb, 0)
    cp.wait()

    lanes = lax.iota(jnp.int32, _L)
    lane_off = lanes * _NF
    pix0 = base + lanes

    def body(i, carry):
        for u in range(8):
            j = i * 8 + u
            t = tri_v[pl.ds(j * _L, _L)]
            plsc.store_scatter(acc_v, [lane_off + t], pix0 + j * _L)
        return carry

    lax.fori_loop(0, _CHUNK // (8 * _L), body, 0)

    def redb(i, carry):
        for u in range(2):
            j = i * 2 + u
            m = acc_v[pl.ds(j * _L, _L)]
            for l in range(1, _L):
                m = jnp.maximum(m, acc_v[pl.ds(l * _NF + j * _L, _L)])
            red_v[pl.ds(j * _L, _L)] = m
        return carry

    lax.fori_loop(0, _NF // (2 * _L), redb, 0)
    pltpu.sync_copy(red_v, out_hbm.at[pl.ds(wid * _NF, _NF)])


_facemax_k = pl.kernel(
    _facemax_body,
    out_type=jax.ShapeDtypeStruct((_NT * _NF,), jnp.int32),
    mesh=_mesh,
    compiler_params=_sc_params,
    scratch_types=[
        pltpu.VMEM((_CHUNK,), jnp.int32),
        pltpu.VMEM((_L * _NF,), jnp.int32),
        pltpu.VMEM((_NF,), jnp.int32),
        pltpu.SemaphoreType.DMA,
    ],
)


_FCS = _NF // 16
def _attn_group_rows(stage, zvecs, i0, n_rows):
    fvecs = []
    for r in range(n_rows):
        i = i0 + r
        s = zvecs[i // _L][i % _L]
        fvecs.append(jnp.broadcast_to(s.astype(jnp.float32), (_L,)))

    def fill(c, carry):
        for r in range(n_rows):
            stage[r, pl.ds(c * _L, _L)] = fvecs[r]
        return carry

    lax.fori_loop(0, _NV // _L, fill, 0)
    tail = 2546 + lax.iota(jnp.int32, _L)
    for r in range(n_rows):
        plsc.store_scatter(stage, [jnp.full((_L,), r, jnp.int32), tail],
                           fvecs[r])


def _gather_body(part_hbm, facesT_hbm, feat_hbm, vfeat_hbm, attn_hbm,
                 mbuf, fbuf, acc_v, red_v, kbuf, cells_v, zsel_v,
                 stage, stage2, rows_v, shared, sem):
    s_id = lax.axis_index("s")
    wid = _wid()
    vb = wid * _VC
    fb = s_id * _FCS
    cps = []
    for r in range(_NT):
        cps.append(pltpu.async_copy(
            part_hbm.at[pl.ds(r * _NF + fb, _FCS)],
            mbuf.at[pl.ds(r * _FCS, _FCS)], sem))
    for k in range(3):
        cps.append(pltpu.async_copy(
            facesT_hbm.at[pl.ds(k * _NF + fb, _FCS)],
            fbuf.at[pl.ds(k * _FCS, _FCS)], sem))

    neg1 = jnp.full((_L,), -1, jnp.int32)

    def initb(i, carry):
        for u in range(8):
            acc_v[pl.ds((i * 8 + u) * _L, _L)] = neg1
        return carry

    lax.fori_loop(0, _L * _NVP // (8 * _L), initb, 0)
    for cp in cps:
        cp.wait()

    lanes = lax.iota(jnp.int32, _L)
    lane_off = lanes * _NVP

    def mainb(i, carry):
        m = mbuf[pl.ds(i * _L, _L)]
        for r in range(1, _NT):
            m = jnp.maximum(m, mbuf[pl.ds(r * _FCS + i * _L, _L)])
        valid = m >= 0
        for k in range(3):
            vid = fbuf[pl.ds(k * _FCS + i * _L, _L)]
            key = jnp.where(valid, k * _HW + m, -1)
            idx = lane_off + vid
            old = plsc.load_gather(acc_v, [idx])
            plsc.store_scatter(acc_v, [idx], jnp.maximum(old, key))
        return carry

    lax.fori_loop(0, _FCS // _L, mainb, 0)

    def redb(i, carry):
        m = acc_v[pl.ds(i * _L, _L)]
        for l in range(1, _L):
            m = jnp.maximum(m, acc_v[pl.ds(l * _NVP + i * _L, _L)])
        red_v[pl.ds(i * _L, _L)] = m
        return carry

    lax.fori_loop(0, _NVP // _L, redb, 0)

    pltpu.sync_copy(red_v, shared.at[pl.ds(s_id * _NVP, _NVP)])
    plsc.subcore_barrier()

    cps2 = []
    for t in range(16):
        cps2.append(pltpu.async_copy(
            shared.at[pl.ds(t * _NVP + vb, _VC)],
            kbuf.at[pl.ds(t * _VC, _VC)], sem))
    for cp in cps2:
        cp.wait()

    def cb(i, carry):
        m = kbuf[pl.ds(i * _L, _L)]
        for r in range(1, 16):
            m = jnp.maximum(m, kbuf[pl.ds(r * _VC + i * _L, _L)])
        valid = m >= 0
        pix = lax.rem(m, _HW)
        ii = lax.div(pix, _W)
        jj = lax.rem(pix, _W)
        cell = lax.div(ii, 8) * 160 + lax.div(jj, 8)
        cells_v[pl.ds(i * _L, _L)] = jnp.where(valid, cell, _CELLS)
        zsel_v[pl.ds(i * _L, _L)] = jnp.where(valid, 0, 1)
        return carry

    lax.fori_loop(0, _VC // _L, cb, 0)

    g = pltpu.async_copy(feat_hbm.at[cells_v], rows_v, sem)

    zvecs = [zsel_v[pl.ds(b * _L, _L)] for b in range(_VC // _L)]

    @pl.when(vb + _VC <= _NV)
    def _attn_full():
        for grp in range(_VC // 8):
            _attn_group_rows(stage, zvecs, 8 * grp, 8)
            pltpu.sync_copy(stage, attn_hbm.at[pl.ds(vb + 8 * grp, 8)])

    @pl.when(jnp.logical_and(vb < _NV, vb + _VC > _NV))
    def _attn_tail():
        for grp in range((_NV % _VC) // 8):
            _attn_group_rows(stage, zvecs, 8 * grp, 8)
            pltpu.sync_copy(stage, attn_hbm.at[pl.ds(vb + 8 * grp, 8)])
        _attn_group_rows(stage2, zvecs, (_NV % _VC) // 8 * 8, 2)
        pltpu.sync_copy(stage2, attn_hbm.at[pl.ds(_NV - 2, 2)])

    g.wait()
    pltpu.sync_copy(rows_v, vfeat_hbm.at[pl.ds(vb, _VC)])


_gather_k = pl.kernel(
    _gather_body,
    out_type=(
        jax.ShapeDtypeStruct((_NVP, _C), jnp.float32),
        jax.ShapeDtypeStruct((_NV, _NV), jnp.float32),
    ),
    mesh=_mesh,
    compiler_params=_sc_params,
    scratch_types=[
        pltpu.VMEM((_NT * _FCS,), jnp.int32),
        pltpu.VMEM((3 * _FCS,), jnp.int32),
        pltpu.VMEM((_L * _NVP,), jnp.int32),
        pltpu.VMEM((_NVP,), jnp.int32),
        pltpu.VMEM((16 * _VC,), jnp.int32),
        pltpu.VMEM((_VC,), jnp.int32),
        pltpu.VMEM((_VC,), jnp.int32),
        pltpu.VMEM((8, _NV), jnp.float32),
        pltpu.VMEM((2, _NV), jnp.float32),
        pltpu.VMEM((_VC, _C), jnp.float32),
        pltpu.VMEM_SHARED((16 * _NVP,), jnp.int32),
        pltpu.SemaphoreType.DMA,
    ],
)


def _transpose_body(x_ref, o_ref):
    i = pl.program_id(0)
    x = x_ref[...]
    xt = jnp.transpose(x)
    rows = lax.broadcasted_iota(jnp.int32, (1920, 1), 0) + i * 1920
    o_ref[...] = jnp.where(rows < _CELLS, xt, 0.0)


def _transpose_call(img2d):
    return pl.pallas_call(
        _transpose_body,
        grid=(_TBL // 1920,),
        in_specs=[pl.BlockSpec((_C, 1920), lambda i: (0, i))],
        out_specs=pl.BlockSpec((1920, _C), lambda i: (i, 0)),
        out_shape=jax.ShapeDtypeStruct((_TBL, _C), jnp.float32),
    )(img2d)


def kernel(rgb_filename, vertices_mesh, faces_mesh, cam_extrinsics,
           intrinsics_mat, image_features, tri_ids):
    tri_flat = tri_ids.reshape(-1).astype(jnp.int32)
    facesT = jnp.transpose(faces_mesh[0]).reshape(-1).astype(jnp.int32)
    img2d = image_features.reshape(_C, _CELLS)

    partials = _facemax_k(tri_flat)
    feat_pad = _transpose_call(img2d)
    vfeat, attn_mask = _gather_k(partials, facesT, feat_pad)

    return attn_mask, vfeat[:_NV][None]


# speedup vs baseline: 1.2557x; 1.0006x over previous
import functools

import jax
import jax.numpy as jnp
from jax import lax
from jax.experimental import pallas as pl
from jax.experimental.pallas import tpu as pltpu
from jax.experimental.pallas import tpu_sc as plsc

_NV = 2562
_NF = 5120
_H, _W = 720, 1280
_HW = _H * _W
_C = 256
_CELLS = 90 * 160
_TBL = 15360

_NT = 32
_L = 16
_CHUNK = _HW // _NT
_NVP = 3072
_VC = _NVP // _NT

_mesh = plsc.VectorSubcoreMesh(core_axis_name="c", subcore_axis_name="s")
_sc_params = pltpu.CompilerParams(needs_layout_passes=False)


def _wid():
    return lax.axis_index("s") * 2 + lax.axis_index("c")


def _facemax_body(tri_hbm, out_hbm, tri_v, acc_v, red_v, sem):
    wid = _wid()
    base = wid * _CHUNK
    cp = pltpu.async_copy(tri_hbm.at[pl.ds(base, _CHUNK)], tri_v, sem)
    neg1 = jnp.full((_L,), -1, jnp.int32)

    def initb(i, carry):
        for u in range(8):
            acc_v[pl.ds((i * 8 + u) * _L, _L)] = neg1
        return carry

    lax.fori_loop(0, _L * _NF // (8 * _L), initb, 0)
    cp.wait()

    lanes = lax.iota(jnp.int32, _L)
    lane_off = lanes * _NF
    pix0 = base + lanes

    def body(i, carry):
        for u in range(8):
            j = i * 8 + u
            t = tri_v[pl.ds(j * _L, _L)]
            plsc.store_scatter(acc_v, [lane_off + t], pix0 + j * _L)
        return carry

    lax.fori_loop(0, _CHUNK // (8 * _L), body, 0)

    def redb(i, carry):
        for u in range(2):
            j = i * 2 + u
            m = acc_v[pl.ds(j * _L, _L)]
            for l in range(1, _L):
                m = jnp.maximum(m, acc_v[pl.ds(l * _NF + j * _L, _L)])
            red_v[pl.ds(j * _L, _L)] = m
        return carry

    lax.fori_loop(0, _NF // (2 * _L), redb, 0)
    pltpu.sync_copy(red_v, out_hbm.at[pl.ds(wid * _NF, _NF)])


_facemax_k = pl.kernel(
    _facemax_body,
    out_type=jax.ShapeDtypeStruct((_NT * _NF,), jnp.int32),
    mesh=_mesh,
    compiler_params=_sc_params,
    scratch_types=[
        pltpu.VMEM((_CHUNK,), jnp.int32),
        pltpu.VMEM((_L * _NF,), jnp.int32),
        pltpu.VMEM((_NF,), jnp.int32),
        pltpu.SemaphoreType.DMA,
    ],
)


_FCS = _NF // 16
def _attn_group_rows(stage, zvecs, i0, n_rows):
    fvecs = []
    for r in range(n_rows):
        i = i0 + r
        s = zvecs[i // _L][i % _L]
        fvecs.append(jnp.broadcast_to(s.astype(jnp.float32), (_L,)))

    def fill(c, carry):
        for r in range(n_rows):
            stage[r, pl.ds(c * _L, _L)] = fvecs[r]
        return carry

    lax.fori_loop(0, _NV // _L, fill, 0)
    tail = 2546 + lax.iota(jnp.int32, _L)
    for r in range(n_rows):
        plsc.store_scatter(stage, [jnp.full((_L,), r, jnp.int32), tail],
                           fvecs[r])


def _gather_body(part_hbm, facesT_hbm, feat_hbm, vfeat_hbm, attn_hbm,
                 mbuf, fbuf, acc_v, red_v, kbuf, cells_v, zsel_v,
                 stage, stage2, rows_v, shared, sem):
    s_id = lax.axis_index("s")
    wid = _wid()
    vb = wid * _VC
    fb = s_id * _FCS
    cps = []
    for r in range(_NT):
        cps.append(pltpu.async_copy(
            part_hbm.at[pl.ds(r * _NF + fb, _FCS)],
            mbuf.at[pl.ds(r * _FCS, _FCS)], sem))
    for k in range(3):
        cps.append(pltpu.async_copy(
            facesT_hbm.at[pl.ds(k * _NF + fb, _FCS)],
            fbuf.at[pl.ds(k * _FCS, _FCS)], sem))

    neg1 = jnp.full((_L,), -1, jnp.int32)

    def initb(i, carry):
        for u in range(8):
            acc_v[pl.ds((i * 8 + u) * _L, _L)] = neg1
        return carry

    lax.fori_loop(0, _L * _NVP // (8 * _L), initb, 0)
    for cp in cps:
        cp.wait()

    lanes = lax.iota(jnp.int32, _L)
    lane_off = lanes * _NVP

    def mainb(i, carry):
        m = mbuf[pl.ds(i * _L, _L)]
        for r in range(1, _NT):
            m = jnp.maximum(m, mbuf[pl.ds(r * _FCS + i * _L, _L)])
        valid = m >= 0
        for k in range(3):
            vid = fbuf[pl.ds(k * _FCS + i * _L, _L)]
            key = jnp.where(valid, k * _HW + m, -1)
            idx = lane_off + vid
            old = plsc.load_gather(acc_v, [idx])
            plsc.store_scatter(acc_v, [idx], jnp.maximum(old, key))
        return carry

    lax.fori_loop(0, _FCS // _L, mainb, 0)

    def redb(i, carry):
        m = acc_v[pl.ds(i * _L, _L)]
        for l in range(1, _L):
            m = jnp.maximum(m, acc_v[pl.ds(l * _NVP + i * _L, _L)])
        red_v[pl.ds(i * _L, _L)] = m
        return carry

    lax.fori_loop(0, _NVP // _L, redb, 0)

    pltpu.sync_copy(red_v, shared.at[pl.ds(s_id * _NVP, _NVP)])
    plsc.subcore_barrier()

    cps2 = []
    for t in range(16):
        cps2.append(pltpu.async_copy(
            shared.at[pl.ds(t * _NVP + vb, _VC)],
            kbuf.at[pl.ds(t * _VC, _VC)], sem))
    for cp in cps2:
        cp.wait()

    def cb(i, carry):
        m = kbuf[pl.ds(i * _L, _L)]
        for r in range(1, 16):
            m = jnp.maximum(m, kbuf[pl.ds(r * _VC + i * _L, _L)])
        valid = m >= 0
        pix = lax.rem(m, _HW)
        ii = lax.div(pix, _W)
        jj = lax.rem(pix, _W)
        cell = lax.div(ii, 8) * 160 + lax.div(jj, 8)
        cells_v[pl.ds(i * _L, _L)] = jnp.where(valid, cell, _CELLS)
        zsel_v[pl.ds(i * _L, _L)] = jnp.where(valid, 0, 1)
        return carry

    lax.fori_loop(0, _VC // _L, cb, 0)

    g = pltpu.async_copy(feat_hbm.at[cells_v], rows_v, sem)

    zvecs = [zsel_v[pl.ds(b * _L, _L)] for b in range(_VC // _L)]

    @pl.when(vb + _VC <= _NV)
    def _attn_full():
        for grp in range(_VC // 8):
            _attn_group_rows(stage, zvecs, 8 * grp, 8)
            pltpu.sync_copy(stage, attn_hbm.at[pl.ds(vb + 8 * grp, 8)])

    @pl.when(jnp.logical_and(vb < _NV, vb + _VC > _NV))
    def _attn_tail():
        for grp in range((_NV % _VC) // 8):
            _attn_group_rows(stage, zvecs, 8 * grp, 8)
            pltpu.sync_copy(stage, attn_hbm.at[pl.ds(vb + 8 * grp, 8)])
        _attn_group_rows(stage2, zvecs, (_NV % _VC) // 8 * 8, 2)
        pltpu.sync_copy(stage2, attn_hbm.at[pl.ds(_NV - 2, 2)])

    g.wait()
    pltpu.sync_copy(rows_v, vfeat_hbm.at[pl.ds(vb, _VC)])


_gather_k = pl.kernel(
    _gather_body,
    out_type=(
        jax.ShapeDtypeStruct((_NVP, _C), jnp.float32),
        jax.ShapeDtypeStruct((_NV, _NV), jnp.float32),
    ),
    mesh=_mesh,
    compiler_params=_sc_params,
    scratch_types=[
        pltpu.VMEM((_NT * _FCS,), jnp.int32),
        pltpu.VMEM((3 * _FCS,), jnp.int32),
        pltpu.VMEM((_L * _NVP,), jnp.int32),
        pltpu.VMEM((_NVP,), jnp.int32),
        pltpu.VMEM((16 * _VC,), jnp.int32),
        pltpu.VMEM((_VC,), jnp.int32),
        pltpu.VMEM((_VC,), jnp.int32),
        pltpu.VMEM((8, _NV), jnp.float32),
        pltpu.VMEM((2, _NV), jnp.float32),
        pltpu.VMEM((_VC, _C), jnp.float32),
        pltpu.VMEM_SHARED((16 * _NVP,), jnp.int32),
        pltpu.SemaphoreType.DMA,
    ],
)


def _transpose_body(x_ref, o_ref):
    i = pl.program_id(0)
    x = x_ref[...]
    xt = jnp.transpose(x)
    rows = lax.broadcasted_iota(jnp.int32, (1920, 1), 0) + i * 1920
    o_ref[...] = jnp.where(rows < _CELLS, xt, 0.0)


def _transpose_call(img2d):
    return pl.pallas_call(
        _transpose_body,
        grid=(_TBL // 1920,),
        in_specs=[pl.BlockSpec((_C, 1920), lambda i: (0, i))],
        out_specs=pl.BlockSpec((1920, _C), lambda i: (i, 0)),
        out_shape=jax.ShapeDtypeStruct((_TBL, _C), jnp.float32),
    )(img2d)


def kernel(rgb_filename, vertices_mesh, faces_mesh, cam_extrinsics,
           intrinsics_mat, image_features, tri_ids):
    tri_flat = tri_ids.reshape(-1).astype(jnp.int32)
    facesT = jnp.transpose(faces_mesh[0]).reshape(-1).astype(jnp.int32)
    img2d = image_features.reshape(_C, _CELLS)

    partials = _facemax_k(tri_flat)
    feat_pad = _transpose_call(img2d)
    vfeat, attn_mask = _gather_k(partials, facesT, feat_pad)

    return attn_mask, vfeat[:_NV][None]


# speedup vs baseline: 1.2649x; 1.0073x over previous
import functools

import jax
import jax.numpy as jnp
from jax import lax
from jax.experimental import pallas as pl
from jax.experimental.pallas import tpu as pltpu
from jax.experimental.pallas import tpu_sc as plsc

_NV = 2562
_NF = 5120
_H, _W = 720, 1280
_HW = _H * _W
_C = 256
_CELLS = 90 * 160
_TBL = 15360

_NT = 32
_L = 16
_CHUNK = _HW // _NT
_NVP = 3072
_VC = _NVP // _NT

_mesh = plsc.VectorSubcoreMesh(core_axis_name="c", subcore_axis_name="s")
_sc_params = pltpu.CompilerParams(needs_layout_passes=False)


def _wid():
    return lax.axis_index("s") * 2 + lax.axis_index("c")


def _facemax_body(tri_hbm, out_hbm, tri_v, acc_v, red_v, sem):
    wid = _wid()
    base = wid * _CHUNK
    cp = pltpu.async_copy(tri_hbm.at[pl.ds(base, _CHUNK)], tri_v, sem)
    neg1 = jnp.full((_L,), -1, jnp.int32)

    def initb(i, carry):
        for u in range(8):
            acc_v[pl.ds((i * 8 + u) * _L, _L)] = neg1
        return carry

    lax.fori_loop(0, _L * _NF // (8 * _L), initb, 0)
    cp.wait()

    lanes = lax.iota(jnp.int32, _L)
    lane_off = lanes * _NF
    pix0 = base + lanes

    def body(i, carry):
        for u in range(8):
            j = i * 8 + u
            t = tri_v[pl.ds(j * _L, _L)]
            plsc.store_scatter(acc_v, [lane_off + t], pix0 + j * _L)
        return carry

    lax.fori_loop(0, _CHUNK // (8 * _L), body, 0)

    def redb(i, carry):
        for u in range(2):
            j = i * 2 + u
            m = acc_v[pl.ds(j * _L, _L)]
            for l in range(1, _L):
                m = jnp.maximum(m, acc_v[pl.ds(l * _NF + j * _L, _L)])
            red_v[pl.ds(j * _L, _L)] = m
        return carry

    lax.fori_loop(0, _NF // (2 * _L), redb, 0)
    pltpu.sync_copy(red_v, out_hbm.at[pl.ds(wid * _NF, _NF)])


_facemax_k = pl.kernel(
    _facemax_body,
    out_type=jax.ShapeDtypeStruct((_NT * _NF,), jnp.int32),
    mesh=_mesh,
    compiler_params=_sc_params,
    scratch_types=[
        pltpu.VMEM((_CHUNK,), jnp.int32),
        pltpu.VMEM((_L * _NF,), jnp.int32),
        pltpu.VMEM((_NF,), jnp.int32),
        pltpu.SemaphoreType.DMA,
    ],
)


_FCS = _NF // 16
def _fill_const(stagebuf, n_rows, vec):
    def fill(c, carry):
        for r in range(n_rows):
            stagebuf[r, pl.ds(c * _L, _L)] = vec
        return carry

    lax.fori_loop(0, _NV // _L, fill, 0)
    tail = 2546 + lax.iota(jnp.int32, _L)
    for r in range(n_rows):
        plsc.store_scatter(stagebuf, [jnp.full((_L,), r, jnp.int32), tail],
                           vec)


def _attn_group_rows(stage, zvecs, i0, n_rows):
    fvecs = []
    for r in range(n_rows):
        i = i0 + r
        s = zvecs[i // _L][i % _L]
        fvecs.append(jnp.broadcast_to(s.astype(jnp.float32), (_L,)))

    def fill(c, carry):
        for r in range(n_rows):
            stage[r, pl.ds(c * _L, _L)] = fvecs[r]
        return carry

    lax.fori_loop(0, _NV // _L, fill, 0)
    tail = 2546 + lax.iota(jnp.int32, _L)
    for r in range(n_rows):
        plsc.store_scatter(stage, [jnp.full((_L,), r, jnp.int32), tail],
                           fvecs[r])


def _gather_body(part_hbm, facesT_hbm, feat_hbm, vfeat_hbm, attn_hbm,
                 mbuf, fbuf, acc_v, red_v, kbuf, cells_v, zsel_v,
                 stage, stage2, rows_v, shared, sem, asem):
    s_id = lax.axis_index("s")
    wid = _wid()
    vb = wid * _VC
    fb = s_id * _FCS
    cps = []
    for r in range(_NT):
        cps.append(pltpu.async_copy(
            part_hbm.at[pl.ds(r * _NF + fb, _FCS)],
            mbuf.at[pl.ds(r * _FCS, _FCS)], sem))
    for k in range(3):
        cps.append(pltpu.async_copy(
            facesT_hbm.at[pl.ds(k * _NF + fb, _FCS)],
            fbuf.at[pl.ds(k * _FCS, _FCS)], sem))

    neg1 = jnp.full((_L,), -1, jnp.int32)

    def initb(i, carry):
        for u in range(8):
            acc_v[pl.ds((i * 8 + u) * _L, _L)] = neg1
        return carry

    lax.fori_loop(0, _L * _NVP // (8 * _L), initb, 0)
    for cp in cps:
        cp.wait()

    lanes = lax.iota(jnp.int32, _L)
    lane_off = lanes * _NVP

    def mainb(i, carry):
        m = mbuf[pl.ds(i * _L, _L)]
        for r in range(1, _NT):
            m = jnp.maximum(m, mbuf[pl.ds(r * _FCS + i * _L, _L)])
        valid = m >= 0
        for k in range(3):
            vid = fbuf[pl.ds(k * _FCS + i * _L, _L)]
            key = jnp.where(valid, k * _HW + m, -1)
            idx = lane_off + vid
            old = plsc.load_gather(acc_v, [idx])
            plsc.store_scatter(acc_v, [idx], jnp.maximum(old, key))
        return carry

    lax.fori_loop(0, _FCS // _L, mainb, 0)

    def redb(i, carry):
        m = acc_v[pl.ds(i * _L, _L)]
        for l in range(1, _L):
            m = jnp.maximum(m, acc_v[pl.ds(l * _NVP + i * _L, _L)])
        red_v[pl.ds(i * _L, _L)] = m
        return carry

    lax.fori_loop(0, _NVP // _L, redb, 0)

    pltpu.sync_copy(red_v, shared.at[pl.ds(s_id * _NVP, _NVP)])
    plsc.subcore_barrier()

    cps2 = []
    for t in range(16):
        cps2.append(pltpu.async_copy(
            shared.at[pl.ds(t * _NVP + vb, _VC)],
            kbuf.at[pl.ds(t * _VC, _VC)], sem))
    for cp in cps2:
        cp.wait()

    def cb(i, carry):
        m = kbuf[pl.ds(i * _L, _L)]
        for r in range(1, 16):
            m = jnp.maximum(m, kbuf[pl.ds(r * _VC + i * _L, _L)])
        valid = m >= 0
        pix = lax.rem(m, _HW)
        ii = lax.div(pix, _W)
        jj = lax.rem(pix, _W)
        cell = lax.div(ii, 8) * 160 + lax.div(jj, 8)
        cells_v[pl.ds(i * _L, _L)] = jnp.where(valid, cell, _CELLS)
        zsel_v[pl.ds(i * _L, _L)] = jnp.where(valid, 0, 1)
        return carry

    lax.fori_loop(0, _VC // _L, cb, 0)

    g = pltpu.async_copy(feat_hbm.at[cells_v], rows_v, sem)

    zvecs = [zsel_v[pl.ds(b * _L, _L)] for b in range(_VC // _L)]

    flags = []
    for b in range(_VC // _L):
        zv = zvecs[b]
        flags.append(jnp.max(jnp.where(lanes < 8, zv, 0)))
        flags.append(jnp.max(jnp.where(lanes >= 8, zv, 0)))

    zero16 = jnp.zeros((_L,), jnp.float32)
    _fill_const(stage, 8, zero16)
    _fill_const(stage2, 2, zero16)

    n_full = _VC // 8
    n_tailg = (_NV % _VC) // 8

    @pl.when(vb + _VC <= _NV)
    def _attn_full():
        ws = []
        for grp in range(n_full):
            ws.append(pltpu.async_copy(
                stage, attn_hbm.at[pl.ds(vb + 8 * grp, 8)], asem))
        for w in ws:
            w.wait()
        for grp in range(n_full):
            @pl.when(flags[grp] > 0)
            def _rw(grp=grp):
                _attn_group_rows(stage, zvecs, 8 * grp, 8)
                pltpu.sync_copy(stage, attn_hbm.at[pl.ds(vb + 8 * grp, 8)])

    @pl.when(jnp.logical_and(vb < _NV, vb + _VC > _NV))
    def _attn_tail():
        ws = []
        for grp in range(n_tailg):
            ws.append(pltpu.async_copy(
                stage, attn_hbm.at[pl.ds(vb + 8 * grp, 8)], asem))
        ws.append(pltpu.async_copy(
            stage2, attn_hbm.at[pl.ds(_NV - 2, 2)], asem))
        for w in ws:
            w.wait()
        for grp in range(n_tailg):
            @pl.when(flags[grp] > 0)
            def _rw(grp=grp):
                _attn_group_rows(stage, zvecs, 8 * grp, 8)
                pltpu.sync_copy(stage, attn_hbm.at[pl.ds(vb + 8 * grp, 8)])
        tail_i0 = n_tailg * 8
        s0 = zvecs[tail_i0 // _L][tail_i0 % _L]
        s1 = zvecs[(tail_i0 + 1) // _L][(tail_i0 + 1) % _L]

        @pl.when(jnp.maximum(s0, s1) > 0)
        def _rwt():
            _attn_group_rows(stage2, zvecs, tail_i0, 2)
            pltpu.sync_copy(stage2, attn_hbm.at[pl.ds(_NV - 2, 2)])

    g.wait()
    pltpu.sync_copy(rows_v, vfeat_hbm.at[pl.ds(vb, _VC)])


_gather_k = pl.kernel(
    _gather_body,
    out_type=(
        jax.ShapeDtypeStruct((_NVP, _C), jnp.float32),
        jax.ShapeDtypeStruct((_NV, _NV), jnp.float32),
    ),
    mesh=_mesh,
    compiler_params=_sc_params,
    scratch_types=[
        pltpu.VMEM((_NT * _FCS,), jnp.int32),
        pltpu.VMEM((3 * _FCS,), jnp.int32),
        pltpu.VMEM((_L * _NVP,), jnp.int32),
        pltpu.VMEM((_NVP,), jnp.int32),
        pltpu.VMEM((16 * _VC,), jnp.int32),
        pltpu.VMEM((_VC,), jnp.int32),
        pltpu.VMEM((_VC,), jnp.int32),
        pltpu.VMEM((8, _NV), jnp.float32),
        pltpu.VMEM((2, _NV), jnp.float32),
        pltpu.VMEM((_VC, _C), jnp.float32),
        pltpu.VMEM_SHARED((16 * _NVP,), jnp.int32),
        pltpu.SemaphoreType.DMA,
        pltpu.SemaphoreType.DMA,
    ],
)


def _transpose_body(x_ref, o_ref):
    i = pl.program_id(0)
    x = x_ref[...]
    xt = jnp.transpose(x)
    rows = lax.broadcasted_iota(jnp.int32, (1920, 1), 0) + i * 1920
    o_ref[...] = jnp.where(rows < _CELLS, xt, 0.0)


def _transpose_call(img2d):
    return pl.pallas_call(
        _transpose_body,
        grid=(_TBL // 1920,),
        in_specs=[pl.BlockSpec((_C, 1920), lambda i: (0, i))],
        out_specs=pl.BlockSpec((1920, _C), lambda i: (i, 0)),
        out_shape=jax.ShapeDtypeStruct((_TBL, _C), jnp.float32),
    )(img2d)


def kernel(rgb_filename, vertices_mesh, faces_mesh, cam_extrinsics,
           intrinsics_mat, image_features, tri_ids):
    tri_flat = tri_ids.reshape(-1).astype(jnp.int32)
    facesT = jnp.transpose(faces_mesh[0]).reshape(-1).astype(jnp.int32)
    img2d = image_features.reshape(_C, _CELLS)

    partials = _facemax_k(tri_flat)
    feat_pad = _transpose_call(img2d)
    vfeat, attn_mask = _gather_k(partials, facesT, feat_pad)

    return attn_mask, vfeat[:_NV][None]
